# Initial kernel scaffold; baseline (speedup 1.0000x reference)
import jax
import jax.numpy as jnp
from jax.experimental import pallas as pl


def kernel(x, dft_fwd, dft_inv, w0, b0, w1, b1, w2, b2, v2_0, wp_0, bp_0, v2_1, wp_1, bp_1, v2_2, wp_2, bp_2):
    raise NotImplementedError("write your pallas kernel here")


# speedup vs baseline: 8.0236x; 8.0236x over previous
import functools

import jax
import jax.numpy as jnp
from jax.experimental import pallas as pl
from jax.experimental.pallas import tpu as pltpu

_KB = 4
_MODES = 32
_W = 64


def _body(x_ref, f_ref, if_ref, w0_ref, b0_ref,
          v2a_ref, wpa_ref, bpa_ref,
          v2b_ref, wpb_ref, bpb_ref,
          v2c_ref, wpc_ref, bpc_ref,
          w1_ref, b1_ref, w2_ref, b2_ref, o_ref):
    f32 = jnp.float32
    bf16 = jnp.bfloat16
    m = _MODES
    w = _W

    h = jnp.dot(x_ref[...], w0_ref[...], preferred_element_type=f32) + b0_ref[...]

    layers = ((v2a_ref, wpa_ref, bpa_ref, False),
              (v2b_ref, wpb_ref, bpb_ref, False),
              (v2c_ref, wpc_ref, bpc_ref, True))
    for v2_ref, wp_ref, bp_ref, last in layers:
        hb = h.astype(bf16)
        xhat = jnp.dot(f_ref[...], hb, preferred_element_type=f32)
        xhb = xhat.astype(bf16)
        v2 = v2_ref[...]
        outs = []
        for e in range(_KB):
            x2m = jnp.concatenate(
                [xhb[:m, e * w:(e + 1) * w], xhb[m:, e * w:(e + 1) * w]],
                axis=1)
            out2 = jnp.einsum("mc,mco->mo", x2m, v2,
                              preferred_element_type=f32)
            outs.append(jnp.concatenate([out2[:, :w], out2[:, w:]], axis=0))
        out_stack = jnp.concatenate(outs, axis=1).astype(bf16)
        x_spec = jnp.dot(if_ref[...], out_stack, preferred_element_type=f32)
        x_point = jnp.dot(hb, wp_ref[...], preferred_element_type=f32) + bp_ref[...]
        h = x_spec + x_point
        if not last:
            h = jnp.maximum(h, 0.0)

    hb = h.astype(bf16)
    h1 = jnp.dot(hb, w1_ref[...], preferred_element_type=f32) + b1_ref[...]
    h1 = jnp.maximum(h1, 0.0).astype(bf16)
    y = jnp.dot(h1, w2_ref[...], preferred_element_type=f32) + b2_ref[...]
    o_ref[...] = y


def _blockdiag(wmat, k):
    return jnp.kron(jnp.eye(k, dtype=wmat.dtype), wmat)


@jax.jit
def kernel(x, dft_fwd, dft_inv, w0, b0, w1, b1, w2, b2,
           v2_0, wp_0, bp_0, v2_1, wp_1, bp_1, v2_2, wp_2, bp_2):
    bf16 = jnp.bfloat16
    B, s, cin0 = x.shape
    kb = _KB
    G = B // kb

    x4 = (x.reshape(G, kb, s, cin0).transpose(0, 2, 1, 3)
          .reshape(G, s, kb * cin0).astype(bf16))

    f_mat = dft_fwd.astype(bf16)
    if_mat = dft_inv.astype(bf16)
    w0k = _blockdiag(w0, kb).astype(bf16)
    b0k = jnp.tile(b0, (1, kb))
    w1k = _blockdiag(w1, kb).astype(bf16)
    b1k = jnp.tile(b1, (1, kb))
    w2k = _blockdiag(w2, kb).astype(bf16)
    b2k = jnp.tile(b2, (1, kb))

    inputs = [x4, f_mat, if_mat, w0k, b0k]
    for v2, wp, bp in ((v2_0, wp_0, bp_0), (v2_1, wp_1, bp_1),
                       (v2_2, wp_2, bp_2)):
        inputs += [v2.astype(bf16), _blockdiag(wp, kb).astype(bf16),
                   jnp.tile(bp, (1, kb))]
    inputs += [w1k, b1k, w2k, b2k]

    def full(arr):
        shp = tuple(arr.shape)
        return pl.BlockSpec(shp, lambda b, _r=len(shp): (0,) * _r)

    in_specs = [pl.BlockSpec((pl.Squeezed(), s, kb * cin0), lambda b: (b, 0, 0))]
    in_specs += [full(a) for a in inputs[1:]]

    out = pl.pallas_call(
        _body,
        out_shape=jax.ShapeDtypeStruct((G, s, kb), jnp.float32),
        grid=(G,),
        in_specs=in_specs,
        out_specs=pl.BlockSpec((pl.Squeezed(), s, kb), lambda b: (b, 0, 0)),
        compiler_params=pltpu.CompilerParams(
            dimension_semantics=("parallel",),
            vmem_limit_bytes=48 * 1024 * 1024,
        ),
    )(*inputs)

    return out.transpose(0, 2, 1).reshape(B, s, 1)


# speedup vs baseline: 10.4007x; 1.2963x over previous
import functools

import jax
import jax.numpy as jnp
from jax.experimental import pallas as pl
from jax.experimental.pallas import tpu as pltpu

_KB = 4
_MODES = 32
_W = 64


def _body(x_ref, f_ref, ifa_ref, w0_ref,
          v2a_ref, wpa_ref,
          v2b_ref, wpb_ref,
          v2c_ref, wpc_ref,
          w1_ref, b1_ref, w2_ref, b2_ref, o_ref):
    f32 = jnp.float32
    bf16 = jnp.bfloat16
    m = _MODES
    w = _W
    kb = _KB

    h = jnp.dot(x_ref[...], w0_ref[...], preferred_element_type=f32)

    if_aug = ifa_ref[...]
    layers = ((v2a_ref, wpa_ref, False),
              (v2b_ref, wpb_ref, False),
              (v2c_ref, wpc_ref, True))
    for v2_ref, wp_ref, last in layers:
        hb = h.astype(bf16)
        xhat = jnp.dot(f_ref[...], hb, preferred_element_type=f32)
        xb = xhat.astype(bf16)
        a3 = xb.reshape(2 * m, kb, w)
        x3d = jnp.concatenate([a3[:m], a3[m:]], axis=2)
        out3 = jnp.einsum("mec,mco->meo", x3d, v2_ref[...],
                          preferred_element_type=f32)
        top = out3[:, :, :w].reshape(m, kb * w)
        bot = out3[:, :, w:].reshape(m, kb * w)
        out_stack = jnp.concatenate([top, bot], axis=0).astype(bf16)
        lhs = jnp.concatenate([hb, if_aug], axis=1)
        wpk = wp_ref[...]
        rhs = jnp.concatenate([wpk[:kb * w], out_stack, wpk[kb * w:]], axis=0)
        h = jnp.dot(lhs, rhs, preferred_element_type=f32)
        if not last:
            h = jnp.maximum(h, 0.0)

    hb = h.astype(bf16)
    h1 = jnp.dot(hb, w1_ref[...], preferred_element_type=f32) + b1_ref[...]
    h1 = jnp.maximum(h1, 0.0).astype(bf16)
    y = jnp.dot(h1, w2_ref[...], preferred_element_type=f32) + b2_ref[...]
    o_ref[...] = y


def _blockdiag(wmat, k):
    return jnp.kron(jnp.eye(k, dtype=wmat.dtype), wmat)


@jax.jit
def kernel(x, dft_fwd, dft_inv, w0, b0, w1, b1, w2, b2,
           v2_0, wp_0, bp_0, v2_1, wp_1, bp_1, v2_2, wp_2, bp_2):
    bf16 = jnp.bfloat16
    B, s, cin0 = x.shape
    kb = _KB
    G = B // kb

    x4 = (x.reshape(G, kb, s, cin0).transpose(0, 2, 1, 3)
          .reshape(G, s, kb * cin0).astype(bf16))
    ones_col = jnp.ones((G, s, 1), bf16)
    x4 = jnp.concatenate([x4, ones_col], axis=2)

    f_mat = dft_fwd.astype(bf16)
    if_aug = jnp.concatenate(
        [dft_inv, jnp.ones((s, 1), dft_inv.dtype)], axis=1).astype(bf16)
    w0k = jnp.concatenate([_blockdiag(w0, kb), jnp.tile(b0, (1, kb))],
                          axis=0).astype(bf16)
    w1k = _blockdiag(w1, kb).astype(bf16)
    b1k = jnp.tile(b1, (1, kb))
    w2k = _blockdiag(w2, kb).astype(bf16)
    b2k = jnp.tile(b2, (1, kb))

    inputs = [x4, f_mat, if_aug, w0k]
    for v2, wp, bp in ((v2_0, wp_0, bp_0), (v2_1, wp_1, bp_1),
                       (v2_2, wp_2, bp_2)):
        wpk = jnp.concatenate([_blockdiag(wp, kb), jnp.tile(bp, (1, kb))],
                              axis=0)
        inputs += [v2.astype(bf16), wpk.astype(bf16)]
    inputs += [w1k, b1k, w2k, b2k]

    def full(arr):
        shp = tuple(arr.shape)
        return pl.BlockSpec(shp, lambda b, _r=len(shp): (0,) * _r)

    in_specs = [pl.BlockSpec((pl.Squeezed(), s, kb * cin0 + 1),
                             lambda b: (b, 0, 0))]
    in_specs += [full(a) for a in inputs[1:]]

    out = pl.pallas_call(
        _body,
        out_shape=jax.ShapeDtypeStruct((G, s, kb), jnp.float32),
        grid=(G,),
        in_specs=in_specs,
        out_specs=pl.BlockSpec((pl.Squeezed(), s, kb), lambda b: (b, 0, 0)),
        compiler_params=pltpu.CompilerParams(
            dimension_semantics=("parallel",),
            vmem_limit_bytes=48 * 1024 * 1024,
        ),
    )(*inputs)

    return out.transpose(0, 2, 1).reshape(B, s, 1)
